# baseline (speedup 1.0000x reference)
import functools

import jax
import jax.numpy as jnp
from jax import lax
from jax.experimental import pallas as pl
from jax.experimental.pallas import tpu as pltpu
from jax.experimental.pallas import tpu_sc as plsc

NUM_CORES = 2
NUM_SUBCORES = 16
LANES = 16
NUM_WORKERS = NUM_CORES * NUM_SUBCORES

BATCH = 16384
DIM = 64
B_PER_W = BATCH // NUM_WORKERS
GATHER_CHUNK = 128
N_CHUNKS = B_PER_W // GATHER_CHUNK


def _body(users_hbm, items_hbm, utab_hbm, itab_hbm, out_hbm,
          uidx_v, iidx_v, urows_v, irows_v, outb_v, sem):
    wid = lax.axis_index("s") * NUM_CORES + lax.axis_index("c")
    base = wid * B_PER_W

    pltpu.sync_copy(users_hbm.at[pl.ds(base, B_PER_W)], uidx_v)
    pltpu.sync_copy(items_hbm.at[pl.ds(base, B_PER_W)], iidx_v)

    copies = []
    for j in range(N_CHUNKS):
        sl = pl.ds(j * GATHER_CHUNK, GATHER_CHUNK)
        copies.append(pltpu.async_copy(
            utab_hbm.at[uidx_v.at[sl]], urows_v.at[sl], sem))
        copies.append(pltpu.async_copy(
            itab_hbm.at[iidx_v.at[sl]], irows_v.at[sl], sem))
    for c in copies:
        c.wait()

    lane = lax.iota(jnp.int32, LANES)

    def group(g, carry):
        rowbase = g * LANES
        accv = jnp.zeros((LANES,), jnp.float32)
        for r in range(LANES):
            row = rowbase + r
            p = urows_v[row, pl.ds(0, LANES)] * irows_v[row, pl.ds(0, LANES)]
            for cth in range(1, DIM // LANES):
                p = p + (urows_v[row, pl.ds(cth * LANES, LANES)]
                         * irows_v[row, pl.ds(cth * LANES, LANES)])
            accv = jnp.where(lane == r, jnp.sum(p), accv)
        outb_v[pl.ds(rowbase, LANES)] = 1.0 / (1.0 + jnp.exp(-accv))
        return carry

    lax.fori_loop(0, B_PER_W // LANES, group, 0)

    pltpu.sync_copy(outb_v, out_hbm.at[pl.ds(base, B_PER_W)])


@jax.jit
def kernel(users, items, user_table, item_table):
    mesh = plsc.VectorSubcoreMesh(core_axis_name="c", subcore_axis_name="s")
    run = functools.partial(
        pl.kernel,
        mesh=mesh,
        compiler_params=pltpu.CompilerParams(
            needs_layout_passes=False, use_tc_tiling_on_sc=False),
        out_type=jax.ShapeDtypeStruct((BATCH,), jnp.float32),
        scratch_types=[
            pltpu.VMEM((B_PER_W,), jnp.int32),
            pltpu.VMEM((B_PER_W,), jnp.int32),
            pltpu.VMEM((B_PER_W, DIM), jnp.float32),
            pltpu.VMEM((B_PER_W, DIM), jnp.float32),
            pltpu.VMEM((B_PER_W,), jnp.float32),
            pltpu.SemaphoreType.DMA,
        ],
    )(_body)
    return run(users, items, user_table, item_table)


# speedup vs baseline: 1.6553x; 1.6553x over previous
import functools

import jax
import jax.numpy as jnp
from jax import lax
from jax.experimental import pallas as pl
from jax.experimental.pallas import tpu as pltpu
from jax.experimental.pallas import tpu_sc as plsc

NUM_CORES = 2
NUM_SUBCORES = 16
LANES = 16
NUM_WORKERS = NUM_CORES * NUM_SUBCORES

BATCH = 16384
DIM = 64
NROWS = 1000000
NTC = (NROWS + 127) // 128
RTC = (NTC + NUM_WORKERS - 1) // NUM_WORKERS
NB = 4
TRASH0 = BATCH
SROWS = BATCH + 128 * NUM_WORKERS
B_PER_W = BATCH // NUM_WORKERS
NGROUPS = BATCH // LANES


def _phase_a(users_hbm, items_hbm, put_hbm, pit_hbm, ustg_hbm, istg_hbm,
             idx_v, hidx_v, hpos_v, bp_v, blk_v, rowbuf_v, fpos_v, sem):
    wid = lax.axis_index("s") * NUM_CORES + lax.axis_index("c")
    tc0 = wid * RTC
    tcN = jnp.minimum(NTC, tc0 + RTC)
    lo = tc0 * 128
    hi = jnp.minimum(NROWS, tcN * 128)
    lane = lax.iota(jnp.int32, LANES)
    trash = TRASH0 + wid * 128

    def one_table(src_idx_hbm, p_hbm, stg_hbm):
        pltpu.sync_copy(src_idx_hbm, idx_v)

        def scan1(g, cnt):
            v = idx_v[pl.ds(g * LANES, LANES)]
            m = (v >= lo) & (v < hi)
            cs = plsc.cumsum(m.astype(jnp.int32))
            pos = cnt + cs - 1
            plsc.store_scatter(hidx_v, [pos], v, mask=m)
            plsc.store_scatter(hpos_v, [pos], g * LANES + lane, mask=m)
            return cnt + plsc.all_reduce_population_count(m)

        cnt = lax.fori_loop(0, NGROUPS, scan1,
                            jnp.zeros((LANES,), jnp.int32))
        nh = cnt[0]
        ngr1 = lax.shift_right_logical(nh + LANES - 1, 4)

        for k in range(128 // LANES):
            fpos_v[pl.ds(k * LANES, LANES)] = trash + k * LANES + lane

        nblocks = (tcN - tc0 + NB - 1) // NB

        def block_fn(b, fill):
            btc0 = tc0 + b * NB
            for j in range(NB):
                tc = btc0 + j

                @pl.when(tc < tcN)
                def _fetch(tc=tc, j=j):
                    pltpu.async_copy(
                        p_hbm.at[:, pl.ds(tc * 128, 128)],
                        blk_v.at[:, pl.ds(j * 128, 128)], sem).wait()

            blo = btc0 * 128
            bhi = jnp.minimum(hi, (btc0 + NB) * 128)

            def scan2(g, bc):
                hv = hidx_v[pl.ds(g * LANES, LANES)]
                hp = hpos_v[pl.ds(g * LANES, LANES)]
                valid = (g * LANES + lane) < nh
                m2 = valid & (hv >= blo) & (hv < bhi)
                pk = (hv - blo) | lax.shift_left(hp, 9)
                p2 = bc + plsc.cumsum(m2.astype(jnp.int32)) - 1
                plsc.store_scatter(bp_v, [p2], pk, mask=m2)
                return bc + plsc.all_reduce_population_count(m2)

            bc = lax.fori_loop(0, ngr1, scan2,
                               jnp.zeros((LANES,), jnp.int32))
            bn = bc[0]
            ngr2 = lax.shift_right_logical(bn + LANES - 1, 4)

            def ext_g(g3, f):
                do_flush = (f[0] + LANES) > 128

                @pl.when(do_flush)
                def _flush():
                    pltpu.async_copy(rowbuf_v, stg_hbm.at[fpos_v],
                                     sem).wait()
                    for k in range(128 // LANES):
                        fpos_v[pl.ds(k * LANES, LANES)] = (
                            trash + k * LANES + lane)

                f2 = jnp.where(do_flush, jnp.zeros_like(f), f)
                pkv = bp_v[pl.ds(g3 * LANES, LANES)]
                vh = (g3 * LANES + lane) < bn
                cols = pkv & 511
                slots = lax.shift_right_logical(pkv, 9)
                rpos = f2 + lane
                plsc.store_scatter(fpos_v, [rpos], slots, mask=vh)
                for d in range(DIM):
                    dvec = jnp.full((LANES,), d, jnp.int32)
                    vals = plsc.load_gather(blk_v, [dvec, cols])
                    plsc.store_scatter(rowbuf_v, [rpos, dvec], vals,
                                       mask=vh)
                return f2 + plsc.all_reduce_population_count(vh)

            return lax.fori_loop(0, ngr2, ext_g, fill)

        lax.fori_loop(0, nblocks, block_fn, jnp.zeros((LANES,), jnp.int32))
        pltpu.async_copy(rowbuf_v, stg_hbm.at[fpos_v], sem).wait()

    one_table(users_hbm, put_hbm, ustg_hbm)
    one_table(items_hbm, pit_hbm, istg_hbm)


PB_CHUNK = 64


def _phase_b(ustg_hbm, istg_hbm, out_hbm, ub_v, ib_v, outb_v, sem):
    wid = lax.axis_index("s") * NUM_CORES + lax.axis_index("c")
    base = wid * B_PER_W
    lane = lax.iota(jnp.int32, LANES)

    def chunk_fn(c, carry):
        cb = base + c * PB_CHUNK
        cu = pltpu.async_copy(ustg_hbm.at[pl.ds(cb, PB_CHUNK), :], ub_v, sem)
        ci = pltpu.async_copy(istg_hbm.at[pl.ds(cb, PB_CHUNK), :], ib_v, sem)
        cu.wait()
        ci.wait()
        for g in range(PB_CHUNK // LANES):
            accv = jnp.zeros((LANES,), jnp.float32)
            for r in range(LANES):
                row = g * LANES + r
                p = ub_v[row, pl.ds(0, LANES)] * ib_v[row, pl.ds(0, LANES)]
                for cth in range(1, DIM // LANES):
                    p = p + (ub_v[row, pl.ds(cth * LANES, LANES)]
                             * ib_v[row, pl.ds(cth * LANES, LANES)])
                accv = jnp.where(lane == r, jnp.sum(p), accv)
            outb_v[pl.ds(g * LANES, LANES)] = 1.0 / (1.0 + jnp.exp(-accv))
        pltpu.sync_copy(outb_v,
                        out_hbm.at[pl.ds(cb, PB_CHUNK)])
        return carry

    lax.fori_loop(0, B_PER_W // PB_CHUNK, chunk_fn, 0)


@jax.jit
def kernel(users, items, user_table, item_table):
    mesh = plsc.VectorSubcoreMesh(core_axis_name="c", subcore_axis_name="s")
    stg_ty = jax.ShapeDtypeStruct((SROWS, 128), jnp.float32)
    run_a = functools.partial(
        pl.kernel,
        mesh=mesh,
        compiler_params=pltpu.CompilerParams(
            needs_layout_passes=False, use_tc_tiling_on_sc=True),
        out_type=(stg_ty, stg_ty),
        scratch_types=[
            pltpu.VMEM((BATCH,), jnp.int32),
            pltpu.VMEM((BATCH,), jnp.int32),
            pltpu.VMEM((BATCH,), jnp.int32),
            pltpu.VMEM((BATCH,), jnp.int32),
            pltpu.VMEM((DIM, NB * 128), jnp.float32),
            pltpu.VMEM((128, 128), jnp.float32),
            pltpu.VMEM((128,), jnp.int32),
            pltpu.SemaphoreType.DMA,
        ],
    )(_phase_a)
    ustg, istg = run_a(users, items, user_table.T, item_table.T)

    run_b = functools.partial(
        pl.kernel,
        mesh=mesh,
        compiler_params=pltpu.CompilerParams(
            needs_layout_passes=False, use_tc_tiling_on_sc=True),
        out_type=jax.ShapeDtypeStruct((BATCH,), jnp.float32),
        scratch_types=[
            pltpu.VMEM((PB_CHUNK, 128), jnp.float32),
            pltpu.VMEM((PB_CHUNK, 128), jnp.float32),
            pltpu.VMEM((PB_CHUNK,), jnp.float32),
            pltpu.SemaphoreType.DMA,
        ],
    )(_phase_b)
    return run_b(ustg, istg)


# speedup vs baseline: 3.7296x; 2.2531x over previous
import functools

import jax
import jax.numpy as jnp
from jax import lax
from jax.experimental import pallas as pl
from jax.experimental.pallas import tpu as pltpu
from jax.experimental.pallas import tpu_sc as plsc

NUM_CORES = 2
NUM_SUBCORES = 16
LANES = 16
NUM_WORKERS = NUM_CORES * NUM_SUBCORES

BATCH = 16384
DIM = 64
NROWS = 1000000
NTC = (NROWS + 127) // 128
RTC = (NTC + NUM_WORKERS - 1) // NUM_WORKERS
NB = 4
TRASH0 = BATCH
SROWS = BATCH + 128 * NUM_WORKERS
B_PER_W = BATCH // NUM_WORKERS
NGROUPS = BATCH // LANES


def _phase_a(users_hbm, items_hbm, put_hbm, pit_hbm, ustg_hbm, istg_hbm,
             idx_v, hpk_v, blk_v, rowbuf_v, fpos_v, sem_f0, sem_f1,
             sem_w):
    bp_v = idx_v
    wid = lax.axis_index("s") * NUM_CORES + lax.axis_index("c")
    tc0 = wid * RTC
    tcN = jnp.minimum(NTC, tc0 + RTC)
    lo = tc0 * 128
    hi = jnp.minimum(NROWS, tcN * 128)
    lane = lax.iota(jnp.int32, LANES)
    trash = TRASH0 + wid * 128

    def one_table(src_idx_hbm, p_hbm, stg_hbm):
        pltpu.sync_copy(src_idx_hbm, idx_v)

        def scan1(g, cnt):
            v = idx_v[pl.ds(g * LANES, LANES)]
            m = (v >= lo) & (v < hi)
            cs = plsc.cumsum(m.astype(jnp.int32))
            pos = cnt + cs - 1
            pk = (v - lo) | lax.shift_left(g * LANES + lane, 15)
            plsc.store_scatter(hpk_v, [pos], pk, mask=m)
            return cnt + plsc.all_reduce_population_count(m)

        cnt = lax.fori_loop(0, NGROUPS, scan1,
                            jnp.zeros((LANES,), jnp.int32))
        nh = cnt[0]
        ngr1 = lax.shift_right_logical(nh + LANES - 1, 4)

        for k in range(128 // LANES):
            fpos_v[pl.ds(k * LANES, LANES)] = trash + k * LANES + lane

        nblocks = (tcN - tc0 + NB - 1) // NB

        def fire_block(b, psem):
            btc0 = tc0 + b * NB
            boff = pl.multiple_of((b % 2) * (NB * 128), 128)
            for j in range(NB):
                tc = btc0 + j

                @pl.when(tc < tcN)
                def _fetch(tc=tc, j=j):
                    pltpu.async_copy(
                        p_hbm.at[:, pl.ds(tc * 128, 128)],
                        blk_v.at[:, pl.ds(boff + j * 128, 128)], psem)

        def drain_block(b, psem):
            btc0 = tc0 + b * NB
            boff = pl.multiple_of((b % 2) * (NB * 128), 128)
            for j in range(NB):
                tc = btc0 + j

                @pl.when(tc < tcN)
                def _drain(tc=tc, j=j):
                    pltpu.make_async_copy(
                        p_hbm.at[:, pl.ds(tc * 128, 128)],
                        blk_v.at[:, pl.ds(boff + j * 128, 128)],
                        psem).wait()

        fire_block(jnp.int32(0), sem_f0)

        def process_block(b, fill):
            boff = pl.multiple_of((b % 2) * (NB * 128), 128)
            blo_l = b * (NB * 128)

            def scan2(g, bc):
                hpk = hpk_v[pl.ds(g * LANES, LANES)]
                hl = hpk & 32767
                hp = lax.shift_right_logical(hpk, 15)
                valid = (g * LANES + lane) < nh
                m2 = valid & (hl >= blo_l) & (hl < blo_l + NB * 128)
                pk = (hl - blo_l) | lax.shift_left(hp, 9)
                p2 = bc + plsc.cumsum(m2.astype(jnp.int32)) - 1
                plsc.store_scatter(bp_v, [p2], pk, mask=m2)
                return bc + plsc.all_reduce_population_count(m2)

            bc = lax.fori_loop(0, ngr1, scan2,
                               jnp.zeros((LANES,), jnp.int32))
            bn = bc[0]
            ngr2 = lax.shift_right_logical(bn + LANES - 1, 4)

            def ext_g(g3, f):
                do_flush = (f[0] + LANES) > 128

                @pl.when(do_flush)
                def _flush():
                    pltpu.async_copy(rowbuf_v, stg_hbm.at[fpos_v],
                                     sem_w).wait()
                    for k in range(128 // LANES):
                        fpos_v[pl.ds(k * LANES, LANES)] = (
                            trash + k * LANES + lane)

                f2 = jnp.where(do_flush, jnp.zeros_like(f), f)
                pkv = bp_v[pl.ds(g3 * LANES, LANES)]
                vh = (g3 * LANES + lane) < bn
                cols = (pkv & 511) + boff
                slots = lax.shift_right_logical(pkv, 9)
                rpos = f2 + lane
                plsc.store_scatter(fpos_v, [rpos], slots, mask=vh)
                for d in range(DIM):
                    dvec = jnp.full((LANES,), d, jnp.int32)
                    vals = plsc.load_gather(blk_v, [dvec, cols])
                    plsc.store_scatter(rowbuf_v, [rpos, dvec], vals,
                                       mask=vh)
                return f2 + plsc.all_reduce_population_count(vh)

            return lax.fori_loop(0, ngr2, ext_g, fill)

        def pair_fn(k, fill):
            b0 = 2 * k
            b1 = 2 * k + 1
            fire_block(b1, sem_f1)
            drain_block(b0, sem_f0)
            fill = process_block(b0, fill)
            fire_block(b1 + 1, sem_f0)
            drain_block(b1, sem_f1)
            return process_block(b1, fill)

        lax.fori_loop(0, (nblocks + 1) // 2, pair_fn,
                      jnp.zeros((LANES,), jnp.int32))
        pltpu.async_copy(rowbuf_v, stg_hbm.at[fpos_v], sem_w).wait()

    one_table(users_hbm, put_hbm, ustg_hbm)
    one_table(items_hbm, pit_hbm, istg_hbm)


PB_CHUNK = 64


def _phase_b(ustg_hbm, istg_hbm, out_hbm, ub_v, ib_v, outb_v, sem):
    wid = lax.axis_index("s") * NUM_CORES + lax.axis_index("c")
    base = wid * B_PER_W
    lane = lax.iota(jnp.int32, LANES)

    def chunk_fn(c, carry):
        cb = base + c * PB_CHUNK
        cu = pltpu.async_copy(ustg_hbm.at[pl.ds(cb, PB_CHUNK), :], ub_v, sem)
        ci = pltpu.async_copy(istg_hbm.at[pl.ds(cb, PB_CHUNK), :], ib_v, sem)
        cu.wait()
        ci.wait()
        for g in range(PB_CHUNK // LANES):
            accv = jnp.zeros((LANES,), jnp.float32)
            for r in range(LANES):
                row = g * LANES + r
                p = ub_v[row, pl.ds(0, LANES)] * ib_v[row, pl.ds(0, LANES)]
                for cth in range(1, DIM // LANES):
                    p = p + (ub_v[row, pl.ds(cth * LANES, LANES)]
                             * ib_v[row, pl.ds(cth * LANES, LANES)])
                accv = jnp.where(lane == r, jnp.sum(p), accv)
            outb_v[pl.ds(g * LANES, LANES)] = 1.0 / (1.0 + jnp.exp(-accv))
        pltpu.sync_copy(outb_v,
                        out_hbm.at[pl.ds(cb, PB_CHUNK)])
        return carry

    lax.fori_loop(0, B_PER_W // PB_CHUNK, chunk_fn, 0)


@jax.jit
def kernel(users, items, user_table, item_table):
    mesh = plsc.VectorSubcoreMesh(core_axis_name="c", subcore_axis_name="s")
    stg_ty = jax.ShapeDtypeStruct((SROWS, 128), jnp.float32)
    run_a = functools.partial(
        pl.kernel,
        mesh=mesh,
        compiler_params=pltpu.CompilerParams(
            needs_layout_passes=False, use_tc_tiling_on_sc=True),
        out_type=(stg_ty, stg_ty),
        scratch_types=[
            pltpu.VMEM((BATCH,), jnp.int32),
            pltpu.VMEM((BATCH,), jnp.int32),
            pltpu.VMEM((DIM, 2 * NB * 128), jnp.float32),
            pltpu.VMEM((128, 128), jnp.float32),
            pltpu.VMEM((128,), jnp.int32),
            pltpu.SemaphoreType.DMA,
            pltpu.SemaphoreType.DMA,
            pltpu.SemaphoreType.DMA,
        ],
    )(_phase_a)
    ustg, istg = run_a(users, items, user_table.T, item_table.T)

    run_b = functools.partial(
        pl.kernel,
        mesh=mesh,
        compiler_params=pltpu.CompilerParams(
            needs_layout_passes=False, use_tc_tiling_on_sc=True),
        out_type=jax.ShapeDtypeStruct((BATCH,), jnp.float32),
        scratch_types=[
            pltpu.VMEM((PB_CHUNK, 128), jnp.float32),
            pltpu.VMEM((PB_CHUNK, 128), jnp.float32),
            pltpu.VMEM((PB_CHUNK,), jnp.float32),
            pltpu.SemaphoreType.DMA,
        ],
    )(_phase_b)
    return run_b(ustg, istg)


# speedup vs baseline: 3.7462x; 1.0045x over previous
import functools

import jax
import jax.numpy as jnp
from jax import lax
from jax.experimental import pallas as pl
from jax.experimental.pallas import tpu as pltpu
from jax.experimental.pallas import tpu_sc as plsc

NUM_CORES = 2
NUM_SUBCORES = 16
LANES = 16
NUM_WORKERS = NUM_CORES * NUM_SUBCORES

BATCH = 16384
DIM = 64
NROWS = 1000000
NTC = (NROWS + 127) // 128
RTC = (NTC + NUM_WORKERS - 1) // NUM_WORKERS
NB = 4
TRASH0 = BATCH
SROWS = BATCH + 128 * NUM_WORKERS
B_PER_W = BATCH // NUM_WORKERS
NGROUPS = BATCH // LANES


def _phase_a(users_hbm, items_hbm, put_hbm, pit_hbm, ustg_hbm, istg_hbm,
             idx_v, hpk_v, blk_v, rowbuf_v, fpos_v, bcnt_v, sem_f0, sem_f1,
             sem_w):
    bp_v = idx_v
    wid = lax.axis_index("s") * NUM_CORES + lax.axis_index("c")
    tc0 = wid * RTC
    tcN = jnp.minimum(NTC, tc0 + RTC)
    lo = tc0 * 128
    hi = jnp.minimum(NROWS, tcN * 128)
    lane = lax.iota(jnp.int32, LANES)
    trash = TRASH0 + wid * 128

    def one_table(src_idx_hbm, p_hbm, stg_hbm):
        pltpu.sync_copy(src_idx_hbm, idx_v)
        for k in range(64 // LANES):
            bcnt_v[pl.ds(k * LANES, LANES)] = jnp.zeros((LANES,), jnp.int32)

        def scan1(g, cnt):
            v = idx_v[pl.ds(g * LANES, LANES)]
            m = (v >= lo) & (v < hi)
            cs = plsc.cumsum(m.astype(jnp.int32))
            pos = cnt + cs - 1
            pk = (v - lo) | lax.shift_left(g * LANES + lane, 15)
            plsc.store_scatter(hpk_v, [pos], pk, mask=m)
            blkid = lax.shift_right_logical(v - lo, 9)
            plsc.addupdate_scatter(bcnt_v, [blkid],
                                   jnp.ones((LANES,), jnp.int32), mask=m)
            return cnt + plsc.all_reduce_population_count(m)

        def block_nz(b):
            grp = (b >> 4) << 4
            bcv = bcnt_v[pl.ds(pl.multiple_of(grp, 8), LANES)]
            sel = jnp.sum(jnp.where(lane == (b & 15), bcv, 0))
            return sel > 0

        cnt = lax.fori_loop(0, NGROUPS, scan1,
                            jnp.zeros((LANES,), jnp.int32))
        nh = cnt[0]
        ngr1 = lax.shift_right_logical(nh + LANES - 1, 4)

        for k in range(128 // LANES):
            fpos_v[pl.ds(k * LANES, LANES)] = trash + k * LANES + lane

        nblocks = (tcN - tc0 + NB - 1) // NB

        def fire_block(b, psem):
            btc0 = tc0 + b * NB
            boff = pl.multiple_of((b % 2) * (NB * 128), 128)
            nz = block_nz(b)
            for j in range(NB):
                tc = btc0 + j

                @pl.when(nz & (tc < tcN))
                def _fetch(tc=tc, j=j):
                    pltpu.async_copy(
                        p_hbm.at[:, pl.ds(tc * 128, 128)],
                        blk_v.at[:, pl.ds(boff + j * 128, 128)], psem)

        def drain_block(b, psem):
            btc0 = tc0 + b * NB
            boff = pl.multiple_of((b % 2) * (NB * 128), 128)
            nz = block_nz(b)
            for j in range(NB):
                tc = btc0 + j

                @pl.when(nz & (tc < tcN))
                def _drain(tc=tc, j=j):
                    pltpu.make_async_copy(
                        p_hbm.at[:, pl.ds(tc * 128, 128)],
                        blk_v.at[:, pl.ds(boff + j * 128, 128)],
                        psem).wait()

        fire_block(jnp.int32(0), sem_f0)

        def process_block(b, fill):
            boff = pl.multiple_of((b % 2) * (NB * 128), 128)
            blo_l = b * (NB * 128)

            def scan2(g, bc):
                hpk = hpk_v[pl.ds(g * LANES, LANES)]
                hl = hpk & 32767
                hp = lax.shift_right_logical(hpk, 15)
                valid = (g * LANES + lane) < nh
                m2 = valid & (hl >= blo_l) & (hl < blo_l + NB * 128)
                pk = (hl - blo_l) | lax.shift_left(hp, 9)
                p2 = bc + plsc.cumsum(m2.astype(jnp.int32)) - 1
                plsc.store_scatter(bp_v, [p2], pk, mask=m2)
                return bc + plsc.all_reduce_population_count(m2)

            bc = lax.fori_loop(0, ngr1, scan2,
                               jnp.zeros((LANES,), jnp.int32))
            bn = bc[0]
            ngr2 = lax.shift_right_logical(bn + LANES - 1, 4)

            def ext_g(g3, f):
                do_flush = (f[0] + LANES) > 128

                @pl.when(do_flush)
                def _flush():
                    pltpu.async_copy(rowbuf_v, stg_hbm.at[fpos_v],
                                     sem_w).wait()
                    for k in range(128 // LANES):
                        fpos_v[pl.ds(k * LANES, LANES)] = (
                            trash + k * LANES + lane)

                f2 = jnp.where(do_flush, jnp.zeros_like(f), f)
                pkv = bp_v[pl.ds(g3 * LANES, LANES)]
                vh = (g3 * LANES + lane) < bn
                cols = (pkv & 511) + boff
                slots = lax.shift_right_logical(pkv, 9)
                rpos = f2 + lane
                plsc.store_scatter(fpos_v, [rpos], slots, mask=vh)
                for d in range(DIM):
                    dvec = jnp.full((LANES,), d, jnp.int32)
                    vals = plsc.load_gather(blk_v, [dvec, cols])
                    plsc.store_scatter(rowbuf_v, [rpos, dvec], vals,
                                       mask=vh)
                return f2 + plsc.all_reduce_population_count(vh)

            return lax.fori_loop(0, ngr2, ext_g, fill)

        def pair_fn(k, fill):
            b0 = 2 * k
            b1 = 2 * k + 1
            fire_block(b1, sem_f1)
            drain_block(b0, sem_f0)
            fill = process_block(b0, fill)
            fire_block(b1 + 1, sem_f0)
            drain_block(b1, sem_f1)
            return process_block(b1, fill)

        lax.fori_loop(0, (nblocks + 1) // 2, pair_fn,
                      jnp.zeros((LANES,), jnp.int32))
        pltpu.async_copy(rowbuf_v, stg_hbm.at[fpos_v], sem_w).wait()

    one_table(users_hbm, put_hbm, ustg_hbm)
    one_table(items_hbm, pit_hbm, istg_hbm)


PB_CHUNK = 64


def _phase_b(ustg_hbm, istg_hbm, out_hbm, ub_v, ib_v, outb_v, sem):
    wid = lax.axis_index("s") * NUM_CORES + lax.axis_index("c")
    base = wid * B_PER_W
    lane = lax.iota(jnp.int32, LANES)

    def chunk_fn(c, carry):
        cb = base + c * PB_CHUNK
        cu = pltpu.async_copy(ustg_hbm.at[pl.ds(cb, PB_CHUNK), :], ub_v, sem)
        ci = pltpu.async_copy(istg_hbm.at[pl.ds(cb, PB_CHUNK), :], ib_v, sem)
        cu.wait()
        ci.wait()
        for g in range(PB_CHUNK // LANES):
            accv = jnp.zeros((LANES,), jnp.float32)
            for r in range(LANES):
                row = g * LANES + r
                p = ub_v[row, pl.ds(0, LANES)] * ib_v[row, pl.ds(0, LANES)]
                for cth in range(1, DIM // LANES):
                    p = p + (ub_v[row, pl.ds(cth * LANES, LANES)]
                             * ib_v[row, pl.ds(cth * LANES, LANES)])
                accv = jnp.where(lane == r, jnp.sum(p), accv)
            outb_v[pl.ds(g * LANES, LANES)] = 1.0 / (1.0 + jnp.exp(-accv))
        pltpu.sync_copy(outb_v,
                        out_hbm.at[pl.ds(cb, PB_CHUNK)])
        return carry

    lax.fori_loop(0, B_PER_W // PB_CHUNK, chunk_fn, 0)


@jax.jit
def kernel(users, items, user_table, item_table):
    mesh = plsc.VectorSubcoreMesh(core_axis_name="c", subcore_axis_name="s")
    stg_ty = jax.ShapeDtypeStruct((SROWS, 128), jnp.float32)
    run_a = functools.partial(
        pl.kernel,
        mesh=mesh,
        compiler_params=pltpu.CompilerParams(
            needs_layout_passes=False, use_tc_tiling_on_sc=True),
        out_type=(stg_ty, stg_ty),
        scratch_types=[
            pltpu.VMEM((BATCH,), jnp.int32),
            pltpu.VMEM((BATCH,), jnp.int32),
            pltpu.VMEM((DIM, 2 * NB * 128), jnp.float32),
            pltpu.VMEM((128, 128), jnp.float32),
            pltpu.VMEM((128,), jnp.int32),
            pltpu.VMEM((64,), jnp.int32),
            pltpu.SemaphoreType.DMA,
            pltpu.SemaphoreType.DMA,
            pltpu.SemaphoreType.DMA,
        ],
    )(_phase_a)
    ustg, istg = run_a(users, items, user_table.T, item_table.T)

    run_b = functools.partial(
        pl.kernel,
        mesh=mesh,
        compiler_params=pltpu.CompilerParams(
            needs_layout_passes=False, use_tc_tiling_on_sc=True),
        out_type=jax.ShapeDtypeStruct((BATCH,), jnp.float32),
        scratch_types=[
            pltpu.VMEM((PB_CHUNK, 128), jnp.float32),
            pltpu.VMEM((PB_CHUNK, 128), jnp.float32),
            pltpu.VMEM((PB_CHUNK,), jnp.float32),
            pltpu.SemaphoreType.DMA,
        ],
    )(_phase_b)
    return run_b(ustg, istg)


# speedup vs baseline: 3.9917x; 1.0655x over previous
import functools

import jax
import jax.numpy as jnp
from jax import lax
from jax.experimental import pallas as pl
from jax.experimental.pallas import tpu as pltpu
from jax.experimental.pallas import tpu_sc as plsc

NUM_CORES = 2
NUM_SUBCORES = 16
LANES = 16
NUM_WORKERS = NUM_CORES * NUM_SUBCORES

BATCH = 16384
DIM = 64
NROWS = 1000000
NTC = (NROWS + 127) // 128
RTC = (NTC + NUM_WORKERS - 1) // NUM_WORKERS
NB = 4
TRASH0 = BATCH
SROWS = BATCH + 128 * NUM_WORKERS
B_PER_W = BATCH // NUM_WORKERS
NGROUPS = BATCH // LANES


def _phase_a(users_hbm, items_hbm, put_hbm, pit_hbm, ustg_hbm, istg_hbm,
             idx_v, hpk_v, blk_v, rowbuf_v, fpos_v, bcnt_v, sem_f0, sem_f1,
             sem_w):
    bp_v = idx_v
    wid = lax.axis_index("s") * NUM_CORES + lax.axis_index("c")
    tc0 = wid * RTC
    tcN = jnp.minimum(NTC, tc0 + RTC)
    lo = tc0 * 128
    hi = jnp.minimum(NROWS, tcN * 128)
    lane = lax.iota(jnp.int32, LANES)
    trash = TRASH0 + wid * 128

    def one_table(src_idx_hbm, p_hbm, stg_hbm):
        pltpu.sync_copy(src_idx_hbm, idx_v)
        for k in range(256 // LANES):
            bcnt_v[pl.ds(k * LANES, LANES)] = jnp.zeros((LANES,), jnp.int32)

        def scan1(g, cnt):
            v = idx_v[pl.ds(g * LANES, LANES)]
            m = (v >= lo) & (v < hi)
            cs = plsc.cumsum(m.astype(jnp.int32))
            pos = cnt + cs - 1
            pk = (v - lo) | lax.shift_left(g * LANES + lane, 15)
            plsc.store_scatter(hpk_v, [pos], pk, mask=m)
            tcid = lax.shift_right_logical(v - lo, 7)
            plsc.store_scatter(bcnt_v, [tcid],
                               jnp.ones((LANES,), jnp.int32), mask=m)
            return cnt + plsc.all_reduce_population_count(m)

        def tc_nz(t):
            grp = (t >> 4) << 4
            bcv = bcnt_v[pl.ds(pl.multiple_of(grp, 8), LANES)]
            sel = jnp.sum(jnp.where(lane == (t & 15), bcv, 0))
            return sel > 0

        cnt = lax.fori_loop(0, NGROUPS, scan1,
                            jnp.zeros((LANES,), jnp.int32))
        nh = cnt[0]
        ngr1 = lax.shift_right_logical(nh + LANES - 1, 4)

        for k in range(128 // LANES):
            fpos_v[pl.ds(k * LANES, LANES)] = trash + k * LANES + lane

        nblocks = (tcN - tc0 + NB - 1) // NB

        def fire_block(b, psem):
            btc0 = tc0 + b * NB
            boff = pl.multiple_of((b % 2) * (NB * 128), 128)
            for j in range(NB):
                tc = btc0 + j

                @pl.when(tc_nz(b * NB + j) & (tc < tcN))
                def _fetch(tc=tc, j=j):
                    pltpu.async_copy(
                        p_hbm.at[:, pl.ds(tc * 128, 128)],
                        blk_v.at[:, pl.ds(boff + j * 128, 128)], psem)

        def drain_block(b, psem):
            btc0 = tc0 + b * NB
            boff = pl.multiple_of((b % 2) * (NB * 128), 128)
            for j in range(NB):
                tc = btc0 + j

                @pl.when(tc_nz(b * NB + j) & (tc < tcN))
                def _drain(tc=tc, j=j):
                    pltpu.make_async_copy(
                        p_hbm.at[:, pl.ds(tc * 128, 128)],
                        blk_v.at[:, pl.ds(boff + j * 128, 128)],
                        psem).wait()

        fire_block(jnp.int32(0), sem_f0)

        def process_block(b, fill):
            boff = pl.multiple_of((b % 2) * (NB * 128), 128)
            blo_l = b * (NB * 128)

            def scan2(g, bc):
                hpk = hpk_v[pl.ds(g * LANES, LANES)]
                hl = hpk & 32767
                hp = lax.shift_right_logical(hpk, 15)
                valid = (g * LANES + lane) < nh
                m2 = valid & (hl >= blo_l) & (hl < blo_l + NB * 128)
                pk = (hl - blo_l) | lax.shift_left(hp, 9)
                p2 = bc + plsc.cumsum(m2.astype(jnp.int32)) - 1
                plsc.store_scatter(bp_v, [p2], pk, mask=m2)
                return bc + plsc.all_reduce_population_count(m2)

            bc = lax.fori_loop(0, ngr1, scan2,
                               jnp.zeros((LANES,), jnp.int32))
            bn = bc[0]
            ngr2 = lax.shift_right_logical(bn + LANES - 1, 4)

            def ext_g(g3, f):
                do_flush = (f[0] + LANES) > 128

                @pl.when(do_flush)
                def _flush():
                    pltpu.async_copy(rowbuf_v, stg_hbm.at[fpos_v],
                                     sem_w).wait()
                    for k in range(128 // LANES):
                        fpos_v[pl.ds(k * LANES, LANES)] = (
                            trash + k * LANES + lane)

                f2 = jnp.where(do_flush, jnp.zeros_like(f), f)
                pkv = bp_v[pl.ds(g3 * LANES, LANES)]
                vh = (g3 * LANES + lane) < bn
                cols = (pkv & 511) + boff
                slots = lax.shift_right_logical(pkv, 9)
                rpos = f2 + lane
                plsc.store_scatter(fpos_v, [rpos], slots, mask=vh)
                for d in range(DIM):
                    dvec = jnp.full((LANES,), d, jnp.int32)
                    vals = plsc.load_gather(blk_v, [dvec, cols])
                    plsc.store_scatter(rowbuf_v, [rpos, dvec], vals,
                                       mask=vh)
                return f2 + plsc.all_reduce_population_count(vh)

            return lax.fori_loop(0, ngr2, ext_g, fill)

        def pair_fn(k, fill):
            b0 = 2 * k
            b1 = 2 * k + 1
            fire_block(b1, sem_f1)
            drain_block(b0, sem_f0)
            fill = process_block(b0, fill)
            fire_block(b1 + 1, sem_f0)
            drain_block(b1, sem_f1)
            return process_block(b1, fill)

        lax.fori_loop(0, (nblocks + 1) // 2, pair_fn,
                      jnp.zeros((LANES,), jnp.int32))
        pltpu.async_copy(rowbuf_v, stg_hbm.at[fpos_v], sem_w).wait()

    one_table(users_hbm, put_hbm, ustg_hbm)
    one_table(items_hbm, pit_hbm, istg_hbm)


PB_CHUNK = 64


def _phase_b(ustg_hbm, istg_hbm, out_hbm, ub_v, ib_v, outb_v, sem):
    wid = lax.axis_index("s") * NUM_CORES + lax.axis_index("c")
    base = wid * B_PER_W
    lane = lax.iota(jnp.int32, LANES)

    def chunk_fn(c, carry):
        cb = base + c * PB_CHUNK
        cu = pltpu.async_copy(ustg_hbm.at[pl.ds(cb, PB_CHUNK), :], ub_v, sem)
        ci = pltpu.async_copy(istg_hbm.at[pl.ds(cb, PB_CHUNK), :], ib_v, sem)
        cu.wait()
        ci.wait()
        for g in range(PB_CHUNK // LANES):
            accv = jnp.zeros((LANES,), jnp.float32)
            for r in range(LANES):
                row = g * LANES + r
                p = ub_v[row, pl.ds(0, LANES)] * ib_v[row, pl.ds(0, LANES)]
                for cth in range(1, DIM // LANES):
                    p = p + (ub_v[row, pl.ds(cth * LANES, LANES)]
                             * ib_v[row, pl.ds(cth * LANES, LANES)])
                accv = jnp.where(lane == r, jnp.sum(p), accv)
            outb_v[pl.ds(g * LANES, LANES)] = 1.0 / (1.0 + jnp.exp(-accv))
        pltpu.sync_copy(outb_v,
                        out_hbm.at[pl.ds(cb, PB_CHUNK)])
        return carry

    lax.fori_loop(0, B_PER_W // PB_CHUNK, chunk_fn, 0)


@jax.jit
def kernel(users, items, user_table, item_table):
    mesh = plsc.VectorSubcoreMesh(core_axis_name="c", subcore_axis_name="s")
    stg_ty = jax.ShapeDtypeStruct((SROWS, 128), jnp.float32)
    run_a = functools.partial(
        pl.kernel,
        mesh=mesh,
        compiler_params=pltpu.CompilerParams(
            needs_layout_passes=False, use_tc_tiling_on_sc=True),
        out_type=(stg_ty, stg_ty),
        scratch_types=[
            pltpu.VMEM((BATCH,), jnp.int32),
            pltpu.VMEM((BATCH,), jnp.int32),
            pltpu.VMEM((DIM, 2 * NB * 128), jnp.float32),
            pltpu.VMEM((128, 128), jnp.float32),
            pltpu.VMEM((128,), jnp.int32),
            pltpu.VMEM((256,), jnp.int32),
            pltpu.SemaphoreType.DMA,
            pltpu.SemaphoreType.DMA,
            pltpu.SemaphoreType.DMA,
        ],
    )(_phase_a)
    ustg, istg = run_a(users, items, user_table.T, item_table.T)

    run_b = functools.partial(
        pl.kernel,
        mesh=mesh,
        compiler_params=pltpu.CompilerParams(
            needs_layout_passes=False, use_tc_tiling_on_sc=True),
        out_type=jax.ShapeDtypeStruct((BATCH,), jnp.float32),
        scratch_types=[
            pltpu.VMEM((PB_CHUNK, 128), jnp.float32),
            pltpu.VMEM((PB_CHUNK, 128), jnp.float32),
            pltpu.VMEM((PB_CHUNK,), jnp.float32),
            pltpu.SemaphoreType.DMA,
        ],
    )(_phase_b)
    return run_b(ustg, istg)
